# Initial kernel scaffold; baseline (speedup 1.0000x reference)
import jax
import jax.numpy as jnp
from jax.experimental import pallas as pl


def kernel(input_ids, emb_W, W_ih, W_hh, b_ih, b_hh, in_res_W, q_W, q_b, k_W, k_b, gate_W, gate_b, h2e_W, h2e_b, out_bias, memory_scale, resid_lambda, input_lambda):
    raise NotImplementedError("write your pallas kernel here")


# speedup vs baseline: 4.9833x; 4.9833x over previous
import jax
import jax.numpy as jnp
from jax.experimental import pallas as pl
from jax.experimental.pallas import tpu as pltpu

_EPS_RMS = 1.1920929e-07
_F32_MIN = jnp.finfo(jnp.float32).min


def _embed_body(emb_hbm, ids_ref, wihT_ref, bih_ref, irT_ref,
                xp_ref, ir_ref, scr_ref, sem):
    tile = scr_ref.shape[0]
    base = pl.program_id(0) * tile

    def issue(i, carry):
        row = ids_ref[base + i]
        pltpu.make_async_copy(emb_hbm.at[row], scr_ref.at[i], sem).start()
        return carry

    jax.lax.fori_loop(0, tile, issue, 0)

    def drain(i, carry):
        pltpu.make_async_copy(emb_hbm.at[0], scr_ref.at[0], sem).wait()
        return carry

    jax.lax.fori_loop(0, tile, drain, 0)

    emb = scr_ref[...]
    xp_ref[...] = jnp.dot(emb, wihT_ref[...],
                          preferred_element_type=jnp.float32) + bih_ref[...]
    ir_ref[...] = jnp.dot(emb, irT_ref[...],
                          preferred_element_type=jnp.float32)


def _embed_call(ids_flat, emb_W, W_ihT, b_ih2, in_resT):
    n_tok = ids_flat.shape[0]
    E = emb_W.shape[1]
    H3 = W_ihT.shape[1]
    Hh = in_resT.shape[1]
    TA = 256
    grid = (n_tok // TA,)
    return pl.pallas_call(
        _embed_body,
        grid=grid,
        in_specs=[
            pl.BlockSpec(memory_space=pl.ANY),
            pl.BlockSpec(memory_space=pltpu.SMEM),
            pl.BlockSpec(memory_space=pltpu.VMEM),
            pl.BlockSpec(memory_space=pltpu.VMEM),
            pl.BlockSpec(memory_space=pltpu.VMEM),
        ],
        out_specs=[
            pl.BlockSpec((TA, H3), lambda g: (g, 0)),
            pl.BlockSpec((TA, Hh), lambda g: (g, 0)),
        ],
        out_shape=[
            jax.ShapeDtypeStruct((n_tok, H3), jnp.float32),
            jax.ShapeDtypeStruct((n_tok, Hh), jnp.float32),
        ],
        scratch_shapes=[
            pltpu.VMEM((TA, E), jnp.float32),
            pltpu.SemaphoreType.DMA,
        ],
        compiler_params=pltpu.CompilerParams(
            dimension_semantics=("parallel",),
            vmem_limit_bytes=40 * 1024 * 1024,
        ),
        name="embed_proj",
    )(emb_W, ids_flat, W_ihT, b_ih2, in_resT)


def _gru_body(xp_ref, whhT_ref, bhh_ref, st_ref):
    Hh = whhT_ref.shape[0]
    S = xp_ref.shape[1]

    def step(t, h):
        hg = jnp.dot(h, whhT_ref[...],
                     preferred_element_type=jnp.float32) + bhh_ref[...]
        xt = xp_ref[0, pl.ds(t, 1), :]
        r = jax.nn.sigmoid(xt[:, :Hh] + hg[:, :Hh])
        z = jax.nn.sigmoid(xt[:, Hh:2 * Hh] + hg[:, Hh:2 * Hh])
        n = jnp.tanh(xt[:, 2 * Hh:] + r * hg[:, 2 * Hh:])
        h_new = (1.0 - z) * n + z * h
        st_ref[0, pl.ds(t, 1), :] = h_new
        return h_new

    jax.lax.fori_loop(0, S, step, jnp.zeros((1, Hh), jnp.float32))


def _gru_call(x_proj, W_hhT, b_hh2):
    B, S, H3 = x_proj.shape
    Hh = W_hhT.shape[0]
    return pl.pallas_call(
        _gru_body,
        grid=(B,),
        in_specs=[
            pl.BlockSpec((1, S, H3), lambda b: (b, 0, 0)),
            pl.BlockSpec(memory_space=pltpu.VMEM),
            pl.BlockSpec(memory_space=pltpu.VMEM),
        ],
        out_specs=pl.BlockSpec((1, S, Hh), lambda b: (b, 0, 0)),
        out_shape=jax.ShapeDtypeStruct((B, S, Hh), jnp.float32),
        compiler_params=pltpu.CompilerParams(
            dimension_semantics=("parallel",),
            vmem_limit_bytes=52 * 1024 * 1024,
        ),
        name="gru_scan",
    )(x_proj, W_hhT, b_hh2)


def _attn_body(st_ref, ir_ref, qWT_ref, qb_ref, kWT_ref, kb_ref,
               gw_ref, h2eT_ref, h2eb_ref, scal_ref,
               gated_ref, proj_ref, mx_ref, q_scr, k_scr, s_scr):
    S = st_ref.shape[1]
    lam_r = scal_ref[0, 0]
    lam_i = scal_ref[0, 1]
    mscale = scal_ref[0, 2]
    gate_b = scal_ref[0, 3]
    inv_sqrt_m = scal_ref[0, 4]

    mx_ref[...] = lam_r * st_ref[0] + lam_i * ir_ref[0]
    ms = jnp.mean(mx_ref[...] * mx_ref[...], axis=-1, keepdims=True)
    mx_ref[...] = mx_ref[...] * jax.lax.rsqrt(ms + _EPS_RMS)

    q_scr[...] = jnp.dot(mx_ref[...], qWT_ref[...],
                         preferred_element_type=jnp.float32) + qb_ref[...]
    k_scr[...] = jnp.dot(mx_ref[...], kWT_ref[...],
                         preferred_element_type=jnp.float32) + kb_ref[...]
    proj_ref[0] = jnp.dot(mx_ref[...], h2eT_ref[...],
                          preferred_element_type=jnp.float32) + h2eb_ref[...]

    row = jax.lax.broadcasted_iota(jnp.int32, (S, S), 0)
    col = jax.lax.broadcasted_iota(jnp.int32, (S, S), 1)
    mask = col < row
    scores = jax.lax.dot_general(
        q_scr[...], k_scr[...], (((1,), (1,)), ((), ())),
        preferred_element_type=jnp.float32) * inv_sqrt_m
    s_scr[...] = jnp.where(mask, scores, _F32_MIN)

    m = jnp.max(s_scr[...], axis=-1, keepdims=True)
    row2 = jax.lax.broadcasted_iota(jnp.int32, (S, S), 0)
    col2 = jax.lax.broadcasted_iota(jnp.int32, (S, S), 1)
    s_scr[...] = jnp.exp(s_scr[...] - m) * jnp.where(col2 < row2, 1.0, 0.0)

    denom = jnp.sum(s_scr[...], axis=-1, keepdims=True)
    gpre = jnp.sum(mx_ref[...] * gw_ref[...], axis=-1, keepdims=True) + gate_b
    gate = jax.nn.sigmoid(gpre) * mscale
    gated_ref[0] = s_scr[...] * (gate / jnp.maximum(denom, 1e-6))


def _attn_call(states, inres, qWT, qb2, kWT, kb2, gate_W, h2eT, h2eb2, scal):
    B, S, Hh = states.shape
    Mm = qWT.shape[1]
    E = h2eT.shape[1]
    return pl.pallas_call(
        _attn_body,
        grid=(B,),
        in_specs=[
            pl.BlockSpec((1, S, Hh), lambda b: (b, 0, 0)),
            pl.BlockSpec((1, S, Hh), lambda b: (b, 0, 0)),
            pl.BlockSpec(memory_space=pltpu.VMEM),
            pl.BlockSpec(memory_space=pltpu.VMEM),
            pl.BlockSpec(memory_space=pltpu.VMEM),
            pl.BlockSpec(memory_space=pltpu.VMEM),
            pl.BlockSpec(memory_space=pltpu.VMEM),
            pl.BlockSpec(memory_space=pltpu.VMEM),
            pl.BlockSpec(memory_space=pltpu.VMEM),
            pl.BlockSpec(memory_space=pltpu.SMEM),
        ],
        out_specs=[
            pl.BlockSpec((1, S, S), lambda b: (b, 0, 0)),
            pl.BlockSpec((1, S, E), lambda b: (b, 0, 0)),
        ],
        out_shape=[
            jax.ShapeDtypeStruct((B, S, S), jnp.float32),
            jax.ShapeDtypeStruct((B, S, E), jnp.float32),
        ],
        scratch_shapes=[
            pltpu.VMEM((S, Hh), jnp.float32),
            pltpu.VMEM((S, Mm), jnp.float32),
            pltpu.VMEM((S, Mm), jnp.float32),
            pltpu.VMEM((S, S), jnp.float32),
        ],
        compiler_params=pltpu.CompilerParams(
            dimension_semantics=("parallel",),
            vmem_limit_bytes=52 * 1024 * 1024,
        ),
        name="mix_attn",
    )(states, inres, qWT, qb2, kWT, kb2, gate_W, h2eT, h2eb2, scal)


def _vocab_body(proj_ref, gated_ref, embW_ref, idc_ref, bias_ref, out_ref):
    S = gated_ref.shape[1]
    TV = embW_ref.shape[0]
    vbase = pl.program_id(1) * TV
    lane = jax.lax.broadcasted_iota(jnp.int32, (S, TV), 1) + vbase
    onehot = jnp.where(idc_ref[0] == lane, 1.0, 0.0)
    base = jax.lax.dot_general(
        proj_ref[0], embW_ref[...], (((1,), (1,)), ((), ())),
        preferred_element_type=jnp.float32)
    scat = jnp.dot(gated_ref[0], onehot, preferred_element_type=jnp.float32)
    out_ref[0] = base + scat + bias_ref[...]


def _vocab_call(proj, gated, emb_W, ids_col, bias2):
    B, S, E = proj.shape
    Vv = emb_W.shape[0]
    TV = 1280
    grid = (B, Vv // TV)
    return pl.pallas_call(
        _vocab_body,
        grid=grid,
        in_specs=[
            pl.BlockSpec((1, S, E), lambda b, v: (b, 0, 0)),
            pl.BlockSpec((1, S, S), lambda b, v: (b, 0, 0)),
            pl.BlockSpec((TV, E), lambda b, v: (v, 0)),
            pl.BlockSpec((1, S, 1), lambda b, v: (b, 0, 0)),
            pl.BlockSpec((1, TV), lambda b, v: (0, v)),
        ],
        out_specs=pl.BlockSpec((1, S, TV), lambda b, v: (b, 0, v)),
        out_shape=jax.ShapeDtypeStruct((B, S, Vv), jnp.float32),
        compiler_params=pltpu.CompilerParams(
            dimension_semantics=("parallel", "arbitrary"),
            vmem_limit_bytes=48 * 1024 * 1024,
        ),
        name="vocab_logits",
    )(proj, gated, emb_W, ids_col, bias2)


def kernel(input_ids, emb_W, W_ih, W_hh, b_ih, b_hh, in_res_W, q_W, q_b,
           k_W, k_b, gate_W, gate_b, h2e_W, h2e_b, out_bias,
           memory_scale, resid_lambda, input_lambda):
    B, S = input_ids.shape
    Hh = W_hh.shape[1]
    E = emb_W.shape[1]
    Mm = q_W.shape[0]

    ids_flat = input_ids.reshape(-1).astype(jnp.int32)
    x_proj2, inres2 = _embed_call(
        ids_flat, emb_W, W_ih.T, b_ih.reshape(1, -1), in_res_W.T)
    x_proj = x_proj2.reshape(B, S, 3 * Hh)
    inres = inres2.reshape(B, S, Hh)

    states = _gru_call(x_proj, W_hh.T, b_hh.reshape(1, -1))

    scal = jnp.stack([
        resid_lambda[0], input_lambda[0],
        jnp.asarray(memory_scale, jnp.float32), gate_b[0],
        jnp.asarray(1.0, jnp.float32) / jnp.sqrt(jnp.asarray(Mm, jnp.float32)),
    ]).reshape(1, 5)
    gated, proj = _attn_call(
        states, inres, q_W.T, q_b.reshape(1, -1), k_W.T, k_b.reshape(1, -1),
        gate_W, h2e_W.T, h2e_b.reshape(1, -1), scal)

    ids_col = input_ids[:, :, None].astype(jnp.int32)
    logits = _vocab_call(proj, gated, emb_W, ids_col, out_bias.reshape(1, -1))
    return logits


# speedup vs baseline: 4.9864x; 1.0006x over previous
import jax
import jax.numpy as jnp
from jax.experimental import pallas as pl
from jax.experimental.pallas import tpu as pltpu

_EPS_RMS = 1.1920929e-07
_F32_MIN = jnp.finfo(jnp.float32).min


def _embed_body(emb_hbm, ids_ref, wihT_ref, bih_ref, irT_ref,
                xp_ref, ir_ref, scr_ref, sem):
    tile = scr_ref.shape[0]
    base = pl.program_id(0) * tile

    def issue(i, carry):
        row = ids_ref[base + i]
        pltpu.make_async_copy(emb_hbm.at[row], scr_ref.at[i], sem).start()
        return carry

    jax.lax.fori_loop(0, tile, issue, 0)

    def drain(i, carry):
        pltpu.make_async_copy(emb_hbm.at[0], scr_ref.at[0], sem).wait()
        return carry

    jax.lax.fori_loop(0, tile, drain, 0)

    emb = scr_ref[...]
    xp_ref[...] = jnp.dot(emb, wihT_ref[...],
                          preferred_element_type=jnp.float32) + bih_ref[...]
    ir_ref[...] = jnp.dot(emb, irT_ref[...],
                          preferred_element_type=jnp.float32)


def _embed_call(ids_flat, emb_W, W_ihT, b_ih2, in_resT):
    n_tok = ids_flat.shape[0]
    E = emb_W.shape[1]
    H3 = W_ihT.shape[1]
    Hh = in_resT.shape[1]
    TA = 256
    grid = (n_tok // TA,)
    return pl.pallas_call(
        _embed_body,
        grid=grid,
        in_specs=[
            pl.BlockSpec(memory_space=pl.ANY),
            pl.BlockSpec(memory_space=pltpu.SMEM),
            pl.BlockSpec(memory_space=pltpu.VMEM),
            pl.BlockSpec(memory_space=pltpu.VMEM),
            pl.BlockSpec(memory_space=pltpu.VMEM),
        ],
        out_specs=[
            pl.BlockSpec((TA, H3), lambda g: (g, 0)),
            pl.BlockSpec((TA, Hh), lambda g: (g, 0)),
        ],
        out_shape=[
            jax.ShapeDtypeStruct((n_tok, H3), jnp.float32),
            jax.ShapeDtypeStruct((n_tok, Hh), jnp.float32),
        ],
        scratch_shapes=[
            pltpu.VMEM((TA, E), jnp.float32),
            pltpu.SemaphoreType.DMA,
        ],
        compiler_params=pltpu.CompilerParams(
            dimension_semantics=("parallel",),
            vmem_limit_bytes=40 * 1024 * 1024,
        ),
        name="embed_proj",
    )(emb_W, ids_flat, W_ihT, b_ih2, in_resT)


def _gru_body(xp_ref, whhT_ref, bhh_ref, st_ref):
    Hh = whhT_ref.shape[0]
    S = xp_ref.shape[1]

    def step(t, h):
        hg = jnp.dot(h.astype(jnp.bfloat16), whhT_ref[...],
                     preferred_element_type=jnp.float32) + bhh_ref[...]
        xt = xp_ref[0, pl.ds(t, 1), :]
        r = jax.nn.sigmoid(xt[:, :Hh] + hg[:, :Hh])
        z = jax.nn.sigmoid(xt[:, Hh:2 * Hh] + hg[:, Hh:2 * Hh])
        n = jnp.tanh(xt[:, 2 * Hh:] + r * hg[:, 2 * Hh:])
        h_new = (1.0 - z) * n + z * h
        st_ref[0, pl.ds(t, 1), :] = h_new
        return h_new

    jax.lax.fori_loop(0, S, step, jnp.zeros((1, Hh), jnp.float32))


def _gru_call(x_proj, W_hhT, b_hh2):
    B, S, H3 = x_proj.shape
    Hh = W_hhT.shape[0]
    return pl.pallas_call(
        _gru_body,
        grid=(B,),
        in_specs=[
            pl.BlockSpec((1, S, H3), lambda b: (b, 0, 0)),
            pl.BlockSpec(memory_space=pltpu.VMEM),
            pl.BlockSpec(memory_space=pltpu.VMEM),
        ],
        out_specs=pl.BlockSpec((1, S, Hh), lambda b: (b, 0, 0)),
        out_shape=jax.ShapeDtypeStruct((B, S, Hh), jnp.float32),
        compiler_params=pltpu.CompilerParams(
            dimension_semantics=("parallel",),
            vmem_limit_bytes=52 * 1024 * 1024,
        ),
        name="gru_scan",
    )(x_proj, W_hhT, b_hh2)


def _attn_body(st_ref, ir_ref, qWT_ref, qb_ref, kWT_ref, kb_ref,
               gw_ref, h2eT_ref, h2eb_ref, scal_ref,
               gated_ref, proj_ref, mx_ref, q_scr, k_scr, s_scr):
    S = st_ref.shape[1]
    lam_r = scal_ref[0, 0]
    lam_i = scal_ref[0, 1]
    mscale = scal_ref[0, 2]
    gate_b = scal_ref[0, 3]
    inv_sqrt_m = scal_ref[0, 4]

    mx_ref[...] = lam_r * st_ref[0] + lam_i * ir_ref[0]
    ms = jnp.mean(mx_ref[...] * mx_ref[...], axis=-1, keepdims=True)
    mx_ref[...] = mx_ref[...] * jax.lax.rsqrt(ms + _EPS_RMS)

    q_scr[...] = jnp.dot(mx_ref[...], qWT_ref[...],
                         preferred_element_type=jnp.float32) + qb_ref[...]
    k_scr[...] = jnp.dot(mx_ref[...], kWT_ref[...],
                         preferred_element_type=jnp.float32) + kb_ref[...]
    proj_ref[0] = jnp.dot(mx_ref[...], h2eT_ref[...],
                          preferred_element_type=jnp.float32) + h2eb_ref[...]

    row = jax.lax.broadcasted_iota(jnp.int32, (S, S), 0)
    col = jax.lax.broadcasted_iota(jnp.int32, (S, S), 1)
    mask = col < row
    scores = jax.lax.dot_general(
        q_scr[...], k_scr[...], (((1,), (1,)), ((), ())),
        preferred_element_type=jnp.float32) * inv_sqrt_m
    s_scr[...] = jnp.where(mask, scores, _F32_MIN)

    m = jnp.max(s_scr[...], axis=-1, keepdims=True)
    row2 = jax.lax.broadcasted_iota(jnp.int32, (S, S), 0)
    col2 = jax.lax.broadcasted_iota(jnp.int32, (S, S), 1)
    s_scr[...] = jnp.exp(s_scr[...] - m) * jnp.where(col2 < row2, 1.0, 0.0)

    denom = jnp.sum(s_scr[...], axis=-1, keepdims=True)
    gpre = jnp.sum(mx_ref[...] * gw_ref[...], axis=-1, keepdims=True) + gate_b
    gate = jax.nn.sigmoid(gpre) * mscale
    gated_ref[0] = s_scr[...] * (gate / jnp.maximum(denom, 1e-6))


def _attn_call(states, inres, qWT, qb2, kWT, kb2, gate_W, h2eT, h2eb2, scal):
    B, S, Hh = states.shape
    Mm = qWT.shape[1]
    E = h2eT.shape[1]
    return pl.pallas_call(
        _attn_body,
        grid=(B,),
        in_specs=[
            pl.BlockSpec((1, S, Hh), lambda b: (b, 0, 0)),
            pl.BlockSpec((1, S, Hh), lambda b: (b, 0, 0)),
            pl.BlockSpec(memory_space=pltpu.VMEM),
            pl.BlockSpec(memory_space=pltpu.VMEM),
            pl.BlockSpec(memory_space=pltpu.VMEM),
            pl.BlockSpec(memory_space=pltpu.VMEM),
            pl.BlockSpec(memory_space=pltpu.VMEM),
            pl.BlockSpec(memory_space=pltpu.VMEM),
            pl.BlockSpec(memory_space=pltpu.VMEM),
            pl.BlockSpec(memory_space=pltpu.SMEM),
        ],
        out_specs=[
            pl.BlockSpec((1, S, S), lambda b: (b, 0, 0)),
            pl.BlockSpec((1, S, E), lambda b: (b, 0, 0)),
        ],
        out_shape=[
            jax.ShapeDtypeStruct((B, S, S), jnp.float32),
            jax.ShapeDtypeStruct((B, S, E), jnp.float32),
        ],
        scratch_shapes=[
            pltpu.VMEM((S, Hh), jnp.float32),
            pltpu.VMEM((S, Mm), jnp.float32),
            pltpu.VMEM((S, Mm), jnp.float32),
            pltpu.VMEM((S, S), jnp.float32),
        ],
        compiler_params=pltpu.CompilerParams(
            dimension_semantics=("parallel",),
            vmem_limit_bytes=52 * 1024 * 1024,
        ),
        name="mix_attn",
    )(states, inres, qWT, qb2, kWT, kb2, gate_W, h2eT, h2eb2, scal)


def _vocab_body(proj_ref, gated_ref, embW_ref, idc_ref, bias_ref, out_ref):
    S = gated_ref.shape[1]
    TV = embW_ref.shape[0]
    vbase = pl.program_id(1) * TV
    lane = jax.lax.broadcasted_iota(jnp.int32, (S, TV), 1) + vbase
    onehot = jnp.where(idc_ref[0] == lane, 1.0, 0.0)
    base = jax.lax.dot_general(
        proj_ref[0], embW_ref[...], (((1,), (1,)), ((), ())),
        preferred_element_type=jnp.float32)
    scat = jnp.dot(gated_ref[0], onehot, preferred_element_type=jnp.float32)
    out_ref[0] = base + scat + bias_ref[...]


def _vocab_call(proj, gated, emb_W, ids_col, bias2):
    B, S, E = proj.shape
    Vv = emb_W.shape[0]
    TV = 1280
    grid = (B, Vv // TV)
    return pl.pallas_call(
        _vocab_body,
        grid=grid,
        in_specs=[
            pl.BlockSpec((1, S, E), lambda b, v: (b, 0, 0)),
            pl.BlockSpec((1, S, S), lambda b, v: (b, 0, 0)),
            pl.BlockSpec((TV, E), lambda b, v: (v, 0)),
            pl.BlockSpec((1, S, 1), lambda b, v: (b, 0, 0)),
            pl.BlockSpec((1, TV), lambda b, v: (0, v)),
        ],
        out_specs=pl.BlockSpec((1, S, TV), lambda b, v: (b, 0, v)),
        out_shape=jax.ShapeDtypeStruct((B, S, Vv), jnp.float32),
        compiler_params=pltpu.CompilerParams(
            dimension_semantics=("parallel", "arbitrary"),
            vmem_limit_bytes=48 * 1024 * 1024,
        ),
        name="vocab_logits",
    )(proj, gated, emb_W, ids_col, bias2)


def kernel(input_ids, emb_W, W_ih, W_hh, b_ih, b_hh, in_res_W, q_W, q_b,
           k_W, k_b, gate_W, gate_b, h2e_W, h2e_b, out_bias,
           memory_scale, resid_lambda, input_lambda):
    B, S = input_ids.shape
    Hh = W_hh.shape[1]
    E = emb_W.shape[1]
    Mm = q_W.shape[0]

    ids_flat = input_ids.reshape(-1).astype(jnp.int32)
    x_proj2, inres2 = _embed_call(
        ids_flat, emb_W, W_ih.T, b_ih.reshape(1, -1), in_res_W.T)
    x_proj = x_proj2.reshape(B, S, 3 * Hh)
    inres = inres2.reshape(B, S, Hh)

    states = _gru_call(x_proj, W_hh.T.astype(jnp.bfloat16),
                       b_hh.reshape(1, -1))

    scal = jnp.stack([
        resid_lambda[0], input_lambda[0],
        jnp.asarray(memory_scale, jnp.float32), gate_b[0],
        jnp.asarray(1.0, jnp.float32) / jnp.sqrt(jnp.asarray(Mm, jnp.float32)),
    ]).reshape(1, 5)
    gated, proj = _attn_call(
        states, inres, q_W.T, q_b.reshape(1, -1), k_W.T, k_b.reshape(1, -1),
        gate_W, h2e_W.T, h2e_b.reshape(1, -1), scal)

    ids_col = input_ids[:, :, None].astype(jnp.int32)
    logits = _vocab_call(proj, gated, emb_W, ids_col, out_bias.reshape(1, -1))
    return logits


# speedup vs baseline: 8.6292x; 1.7306x over previous
import jax
import jax.numpy as jnp
from jax.experimental import pallas as pl
from jax.experimental.pallas import tpu as pltpu

_EPS_RMS = 1.1920929e-07
_F32_MIN = jnp.finfo(jnp.float32).min


def _embed_body(emb_hbm, ids_ref, wihT_ref, bih_ref, irT_ref,
                xp_ref, ir_ref, scr_ref, sem):
    tile = scr_ref.shape[0]
    base = pl.program_id(0) * tile

    def issue(i, carry):
        row = ids_ref[base + i]
        pltpu.make_async_copy(emb_hbm.at[row], scr_ref.at[i], sem).start()
        return carry

    jax.lax.fori_loop(0, tile, issue, 0)

    def drain(i, carry):
        pltpu.make_async_copy(emb_hbm.at[0], scr_ref.at[0], sem).wait()
        return carry

    jax.lax.fori_loop(0, tile, drain, 0)

    emb = scr_ref[...]
    xp_ref[...] = jnp.dot(emb, wihT_ref[...],
                          preferred_element_type=jnp.float32) + bih_ref[...]
    ir_ref[...] = jnp.dot(emb, irT_ref[...],
                          preferred_element_type=jnp.float32)


def _embed_call(ids_flat, emb_W, W_ihT, b_ih2, in_resT):
    n_tok = ids_flat.shape[0]
    E = emb_W.shape[1]
    H3 = W_ihT.shape[1]
    Hh = in_resT.shape[1]
    TA = 256
    grid = (n_tok // TA,)
    return pl.pallas_call(
        _embed_body,
        grid=grid,
        in_specs=[
            pl.BlockSpec(memory_space=pl.ANY),
            pl.BlockSpec(memory_space=pltpu.SMEM),
            pl.BlockSpec(memory_space=pltpu.VMEM),
            pl.BlockSpec(memory_space=pltpu.VMEM),
            pl.BlockSpec(memory_space=pltpu.VMEM),
        ],
        out_specs=[
            pl.BlockSpec((TA, H3), lambda g: (g, 0)),
            pl.BlockSpec((TA, Hh), lambda g: (g, 0)),
        ],
        out_shape=[
            jax.ShapeDtypeStruct((n_tok, H3), jnp.float32),
            jax.ShapeDtypeStruct((n_tok, Hh), jnp.float32),
        ],
        scratch_shapes=[
            pltpu.VMEM((TA, E), jnp.float32),
            pltpu.SemaphoreType.DMA,
        ],
        compiler_params=pltpu.CompilerParams(
            dimension_semantics=("parallel",),
            vmem_limit_bytes=40 * 1024 * 1024,
        ),
        name="embed_proj",
    )(emb_W, ids_flat, W_ihT, b_ih2, in_resT)


def _gru_body(xp_ref, whhT_ref, bhh_ref, st_ref):
    Hh = whhT_ref.shape[0]
    B, S = xp_ref.shape[0], xp_ref.shape[1]

    def step(t, h):
        hg = jnp.dot(h.astype(jnp.bfloat16), whhT_ref[...],
                     preferred_element_type=jnp.float32) + bhh_ref[...]
        xt = xp_ref[:, pl.ds(t, 1), :].reshape(B, 3 * Hh)
        r = jax.nn.sigmoid(xt[:, :Hh] + hg[:, :Hh])
        z = jax.nn.sigmoid(xt[:, Hh:2 * Hh] + hg[:, Hh:2 * Hh])
        n = jnp.tanh(xt[:, 2 * Hh:] + r * hg[:, 2 * Hh:])
        h_new = (1.0 - z) * n + z * h
        st_ref[:, pl.ds(t, 1), :] = h_new.reshape(B, 1, Hh)
        return h_new

    jax.lax.fori_loop(0, S, step, jnp.zeros((B, Hh), jnp.float32))


def _gru_call(x_proj, W_hhT, b_hh2):
    B, S, H3 = x_proj.shape
    Hh = W_hhT.shape[0]
    return pl.pallas_call(
        _gru_body,
        in_specs=[
            pl.BlockSpec(memory_space=pltpu.VMEM),
            pl.BlockSpec(memory_space=pltpu.VMEM),
            pl.BlockSpec(memory_space=pltpu.VMEM),
        ],
        out_specs=pl.BlockSpec(memory_space=pltpu.VMEM),
        out_shape=jax.ShapeDtypeStruct((B, S, Hh), jnp.float32),
        compiler_params=pltpu.CompilerParams(
            vmem_limit_bytes=52 * 1024 * 1024,
        ),
        name="gru_scan",
    )(x_proj, W_hhT, b_hh2)


def _attn_body(st_ref, ir_ref, qWT_ref, qb_ref, kWT_ref, kb_ref,
               gw_ref, h2eT_ref, h2eb_ref, scal_ref,
               gated_ref, proj_ref, mx_ref, q_scr, k_scr, s_scr):
    S = st_ref.shape[1]
    lam_r = scal_ref[0, 0]
    lam_i = scal_ref[0, 1]
    mscale = scal_ref[0, 2]
    gate_b = scal_ref[0, 3]
    inv_sqrt_m = scal_ref[0, 4]

    mx_ref[...] = lam_r * st_ref[0] + lam_i * ir_ref[0]
    ms = jnp.mean(mx_ref[...] * mx_ref[...], axis=-1, keepdims=True)
    mx_ref[...] = mx_ref[...] * jax.lax.rsqrt(ms + _EPS_RMS)

    q_scr[...] = jnp.dot(mx_ref[...], qWT_ref[...],
                         preferred_element_type=jnp.float32) + qb_ref[...]
    k_scr[...] = jnp.dot(mx_ref[...], kWT_ref[...],
                         preferred_element_type=jnp.float32) + kb_ref[...]
    proj_ref[0] = jnp.dot(mx_ref[...], h2eT_ref[...],
                          preferred_element_type=jnp.float32) + h2eb_ref[...]

    row = jax.lax.broadcasted_iota(jnp.int32, (S, S), 0)
    col = jax.lax.broadcasted_iota(jnp.int32, (S, S), 1)
    mask = col < row
    scores = jax.lax.dot_general(
        q_scr[...], k_scr[...], (((1,), (1,)), ((), ())),
        preferred_element_type=jnp.float32) * inv_sqrt_m
    s_scr[...] = jnp.where(mask, scores, _F32_MIN)

    m = jnp.max(s_scr[...], axis=-1, keepdims=True)
    row2 = jax.lax.broadcasted_iota(jnp.int32, (S, S), 0)
    col2 = jax.lax.broadcasted_iota(jnp.int32, (S, S), 1)
    s_scr[...] = jnp.exp(s_scr[...] - m) * jnp.where(col2 < row2, 1.0, 0.0)

    denom = jnp.sum(s_scr[...], axis=-1, keepdims=True)
    gpre = jnp.sum(mx_ref[...] * gw_ref[...], axis=-1, keepdims=True) + gate_b
    gate = jax.nn.sigmoid(gpre) * mscale
    gated_ref[0] = s_scr[...] * (gate / jnp.maximum(denom, 1e-6))


def _attn_call(states, inres, qWT, qb2, kWT, kb2, gate_W, h2eT, h2eb2, scal):
    B, S, Hh = states.shape
    Mm = qWT.shape[1]
    E = h2eT.shape[1]
    return pl.pallas_call(
        _attn_body,
        grid=(B,),
        in_specs=[
            pl.BlockSpec((1, S, Hh), lambda b: (b, 0, 0)),
            pl.BlockSpec((1, S, Hh), lambda b: (b, 0, 0)),
            pl.BlockSpec(memory_space=pltpu.VMEM),
            pl.BlockSpec(memory_space=pltpu.VMEM),
            pl.BlockSpec(memory_space=pltpu.VMEM),
            pl.BlockSpec(memory_space=pltpu.VMEM),
            pl.BlockSpec(memory_space=pltpu.VMEM),
            pl.BlockSpec(memory_space=pltpu.VMEM),
            pl.BlockSpec(memory_space=pltpu.VMEM),
            pl.BlockSpec(memory_space=pltpu.SMEM),
        ],
        out_specs=[
            pl.BlockSpec((1, S, S), lambda b: (b, 0, 0)),
            pl.BlockSpec((1, S, E), lambda b: (b, 0, 0)),
        ],
        out_shape=[
            jax.ShapeDtypeStruct((B, S, S), jnp.float32),
            jax.ShapeDtypeStruct((B, S, E), jnp.float32),
        ],
        scratch_shapes=[
            pltpu.VMEM((S, Hh), jnp.float32),
            pltpu.VMEM((S, Mm), jnp.float32),
            pltpu.VMEM((S, Mm), jnp.float32),
            pltpu.VMEM((S, S), jnp.float32),
        ],
        compiler_params=pltpu.CompilerParams(
            dimension_semantics=("parallel",),
            vmem_limit_bytes=52 * 1024 * 1024,
        ),
        name="mix_attn",
    )(states, inres, qWT, qb2, kWT, kb2, gate_W, h2eT, h2eb2, scal)


def _vocab_body(proj_ref, gated_ref, embW_ref, idc_ref, bias_ref, out_ref):
    S = gated_ref.shape[1]
    TV = embW_ref.shape[0]
    vbase = pl.program_id(1) * TV
    lane = jax.lax.broadcasted_iota(jnp.int32, (S, TV), 1) + vbase
    onehot = jnp.where(idc_ref[0] == lane, 1.0, 0.0)
    base = jax.lax.dot_general(
        proj_ref[0], embW_ref[...], (((1,), (1,)), ((), ())),
        preferred_element_type=jnp.float32)
    scat = jnp.dot(gated_ref[0], onehot, preferred_element_type=jnp.float32)
    out_ref[0] = base + scat + bias_ref[...]


def _vocab_call(proj, gated, emb_W, ids_col, bias2):
    B, S, E = proj.shape
    Vv = emb_W.shape[0]
    TV = 1280
    grid = (B, Vv // TV)
    return pl.pallas_call(
        _vocab_body,
        grid=grid,
        in_specs=[
            pl.BlockSpec((1, S, E), lambda b, v: (b, 0, 0)),
            pl.BlockSpec((1, S, S), lambda b, v: (b, 0, 0)),
            pl.BlockSpec((TV, E), lambda b, v: (v, 0)),
            pl.BlockSpec((1, S, 1), lambda b, v: (b, 0, 0)),
            pl.BlockSpec((1, TV), lambda b, v: (0, v)),
        ],
        out_specs=pl.BlockSpec((1, S, TV), lambda b, v: (b, 0, v)),
        out_shape=jax.ShapeDtypeStruct((B, S, Vv), jnp.float32),
        compiler_params=pltpu.CompilerParams(
            dimension_semantics=("parallel", "arbitrary"),
            vmem_limit_bytes=48 * 1024 * 1024,
        ),
        name="vocab_logits",
    )(proj, gated, emb_W, ids_col, bias2)


def kernel(input_ids, emb_W, W_ih, W_hh, b_ih, b_hh, in_res_W, q_W, q_b,
           k_W, k_b, gate_W, gate_b, h2e_W, h2e_b, out_bias,
           memory_scale, resid_lambda, input_lambda):
    B, S = input_ids.shape
    Hh = W_hh.shape[1]
    E = emb_W.shape[1]
    Mm = q_W.shape[0]

    ids_flat = input_ids.reshape(-1).astype(jnp.int32)
    x_proj2, inres2 = _embed_call(
        ids_flat, emb_W, W_ih.T, b_ih.reshape(1, -1), in_res_W.T)
    x_proj = x_proj2.reshape(B, S, 3 * Hh)
    inres = inres2.reshape(B, S, Hh)

    states = _gru_call(x_proj, W_hh.T.astype(jnp.bfloat16),
                       b_hh.reshape(1, -1))

    scal = jnp.stack([
        resid_lambda[0], input_lambda[0],
        jnp.asarray(memory_scale, jnp.float32), gate_b[0],
        jnp.asarray(1.0, jnp.float32) / jnp.sqrt(jnp.asarray(Mm, jnp.float32)),
    ]).reshape(1, 5)
    gated, proj = _attn_call(
        states, inres, q_W.T, q_b.reshape(1, -1), k_W.T, k_b.reshape(1, -1),
        gate_W, h2e_W.T, h2e_b.reshape(1, -1), scal)

    ids_col = input_ids[:, :, None].astype(jnp.int32)
    logits = _vocab_call(proj, gated, emb_W, ids_col, out_bias.reshape(1, -1))
    return logits


# speedup vs baseline: 8.7521x; 1.0142x over previous
import jax
import jax.numpy as jnp
from jax.experimental import pallas as pl
from jax.experimental.pallas import tpu as pltpu

_EPS_RMS = 1.1920929e-07
_F32_MIN = jnp.finfo(jnp.float32).min


def _embed_body(emb_hbm, ids_ref, wihT_ref, bih_ref, irT_ref,
                xp_ref, ir_ref, scr_ref, sem):
    tile = scr_ref.shape[0]
    base = pl.program_id(0) * tile

    def issue(i, carry):
        row = ids_ref[base + i]
        pltpu.make_async_copy(emb_hbm.at[row], scr_ref.at[i], sem).start()
        return carry

    jax.lax.fori_loop(0, tile, issue, 0)

    def drain(i, carry):
        pltpu.make_async_copy(emb_hbm.at[0], scr_ref.at[0], sem).wait()
        return carry

    jax.lax.fori_loop(0, tile, drain, 0)

    emb = scr_ref[...]
    xp_ref[...] = jnp.dot(emb, wihT_ref[...],
                          preferred_element_type=jnp.float32) + bih_ref[...]
    ir_ref[...] = jnp.dot(emb, irT_ref[...],
                          preferred_element_type=jnp.float32)


def _embed_call(ids_flat, emb_W, W_ihT, b_ih2, in_resT):
    n_tok = ids_flat.shape[0]
    E = emb_W.shape[1]
    H3 = W_ihT.shape[1]
    Hh = in_resT.shape[1]
    TA = 256
    grid = (n_tok // TA,)
    return pl.pallas_call(
        _embed_body,
        grid=grid,
        in_specs=[
            pl.BlockSpec(memory_space=pl.ANY),
            pl.BlockSpec(memory_space=pltpu.SMEM),
            pl.BlockSpec(memory_space=pltpu.VMEM),
            pl.BlockSpec(memory_space=pltpu.VMEM),
            pl.BlockSpec(memory_space=pltpu.VMEM),
        ],
        out_specs=[
            pl.BlockSpec((TA, H3), lambda g: (g, 0)),
            pl.BlockSpec((TA, Hh), lambda g: (g, 0)),
        ],
        out_shape=[
            jax.ShapeDtypeStruct((n_tok, H3), jnp.float32),
            jax.ShapeDtypeStruct((n_tok, Hh), jnp.float32),
        ],
        scratch_shapes=[
            pltpu.VMEM((TA, E), jnp.float32),
            pltpu.SemaphoreType.DMA,
        ],
        compiler_params=pltpu.CompilerParams(
            dimension_semantics=("parallel",),
            vmem_limit_bytes=40 * 1024 * 1024,
        ),
        name="embed_proj",
    )(emb_W, ids_flat, W_ihT, b_ih2, in_resT)


def _gru_body(xp_ref, whhT_ref, bhh_ref, st_ref):
    Hh = whhT_ref.shape[0]
    B, S = xp_ref.shape[0], xp_ref.shape[1]

    def step(t, h):
        hg = jnp.dot(h.astype(jnp.bfloat16), whhT_ref[...],
                     preferred_element_type=jnp.float32) + bhh_ref[...]
        xt = xp_ref[:, pl.ds(t, 1), :].reshape(B, 3 * Hh)
        r = jax.nn.sigmoid(xt[:, :Hh] + hg[:, :Hh])
        z = jax.nn.sigmoid(xt[:, Hh:2 * Hh] + hg[:, Hh:2 * Hh])
        n = jnp.tanh(xt[:, 2 * Hh:] + r * hg[:, 2 * Hh:])
        h_new = (1.0 - z) * n + z * h
        st_ref[:, pl.ds(t, 1), :] = h_new.reshape(B, 1, Hh)
        return h_new

    def step2(i, h):
        h1 = step(2 * i, h)
        return step(2 * i + 1, h1)

    jax.lax.fori_loop(0, S // 2, step2, jnp.zeros((B, Hh), jnp.float32))


def _gru_call(x_proj, W_hhT, b_hh2):
    B, S, H3 = x_proj.shape
    Hh = W_hhT.shape[0]
    return pl.pallas_call(
        _gru_body,
        in_specs=[
            pl.BlockSpec(memory_space=pltpu.VMEM),
            pl.BlockSpec(memory_space=pltpu.VMEM),
            pl.BlockSpec(memory_space=pltpu.VMEM),
        ],
        out_specs=pl.BlockSpec(memory_space=pltpu.VMEM),
        out_shape=jax.ShapeDtypeStruct((B, S, Hh), jnp.float32),
        compiler_params=pltpu.CompilerParams(
            vmem_limit_bytes=52 * 1024 * 1024,
        ),
        name="gru_scan",
    )(x_proj, W_hhT, b_hh2)


def _attn_body(st_ref, ir_ref, qWT_ref, qb_ref, kWT_ref, kb_ref,
               gw_ref, h2eT_ref, h2eb_ref, scal_ref,
               gated_ref, proj_ref, mx_ref, q_scr, k_scr, s_scr):
    S = st_ref.shape[1]
    lam_r = scal_ref[0, 0]
    lam_i = scal_ref[0, 1]
    mscale = scal_ref[0, 2]
    gate_b = scal_ref[0, 3]
    inv_sqrt_m = scal_ref[0, 4]

    mx_ref[...] = lam_r * st_ref[0] + lam_i * ir_ref[0]
    ms = jnp.mean(mx_ref[...] * mx_ref[...], axis=-1, keepdims=True)
    mx_ref[...] = mx_ref[...] * jax.lax.rsqrt(ms + _EPS_RMS)

    q_scr[...] = jnp.dot(mx_ref[...], qWT_ref[...],
                         preferred_element_type=jnp.float32) + qb_ref[...]
    k_scr[...] = jnp.dot(mx_ref[...], kWT_ref[...],
                         preferred_element_type=jnp.float32) + kb_ref[...]
    proj_ref[0] = jnp.dot(mx_ref[...], h2eT_ref[...],
                          preferred_element_type=jnp.float32) + h2eb_ref[...]

    row = jax.lax.broadcasted_iota(jnp.int32, (S, S), 0)
    col = jax.lax.broadcasted_iota(jnp.int32, (S, S), 1)
    mask = col < row
    scores = jax.lax.dot_general(
        q_scr[...], k_scr[...], (((1,), (1,)), ((), ())),
        preferred_element_type=jnp.float32) * inv_sqrt_m
    s_scr[...] = jnp.where(mask, scores, _F32_MIN)

    m = jnp.max(s_scr[...], axis=-1, keepdims=True)
    row2 = jax.lax.broadcasted_iota(jnp.int32, (S, S), 0)
    col2 = jax.lax.broadcasted_iota(jnp.int32, (S, S), 1)
    s_scr[...] = jnp.exp(s_scr[...] - m) * jnp.where(col2 < row2, 1.0, 0.0)

    denom = jnp.sum(s_scr[...], axis=-1, keepdims=True)
    gpre = jnp.sum(mx_ref[...] * gw_ref[...], axis=-1, keepdims=True) + gate_b
    gate = jax.nn.sigmoid(gpre) * mscale
    gated_ref[0] = (s_scr[...] * (gate / jnp.maximum(denom, 1e-6))
                    ).astype(jnp.bfloat16)


def _attn_call(states, inres, qWT, qb2, kWT, kb2, gate_W, h2eT, h2eb2, scal):
    B, S, Hh = states.shape
    Mm = qWT.shape[1]
    E = h2eT.shape[1]
    return pl.pallas_call(
        _attn_body,
        grid=(B,),
        in_specs=[
            pl.BlockSpec((1, S, Hh), lambda b: (b, 0, 0)),
            pl.BlockSpec((1, S, Hh), lambda b: (b, 0, 0)),
            pl.BlockSpec(memory_space=pltpu.VMEM),
            pl.BlockSpec(memory_space=pltpu.VMEM),
            pl.BlockSpec(memory_space=pltpu.VMEM),
            pl.BlockSpec(memory_space=pltpu.VMEM),
            pl.BlockSpec(memory_space=pltpu.VMEM),
            pl.BlockSpec(memory_space=pltpu.VMEM),
            pl.BlockSpec(memory_space=pltpu.VMEM),
            pl.BlockSpec(memory_space=pltpu.SMEM),
        ],
        out_specs=[
            pl.BlockSpec((1, S, S), lambda b: (b, 0, 0)),
            pl.BlockSpec((1, S, E), lambda b: (b, 0, 0)),
        ],
        out_shape=[
            jax.ShapeDtypeStruct((B, S, S), jnp.bfloat16),
            jax.ShapeDtypeStruct((B, S, E), jnp.float32),
        ],
        scratch_shapes=[
            pltpu.VMEM((S, Hh), jnp.float32),
            pltpu.VMEM((S, Mm), jnp.float32),
            pltpu.VMEM((S, Mm), jnp.float32),
            pltpu.VMEM((S, S), jnp.float32),
        ],
        compiler_params=pltpu.CompilerParams(
            dimension_semantics=("parallel",),
            vmem_limit_bytes=52 * 1024 * 1024,
        ),
        name="mix_attn",
    )(states, inres, qWT, qb2, kWT, kb2, gate_W, h2eT, h2eb2, scal)


def _vocab_body(proj_ref, gated_ref, embW_ref, idc_ref, bias_ref, out_ref):
    S = gated_ref.shape[1]
    TV = embW_ref.shape[0]
    vbase = pl.program_id(1) * TV
    lane = jax.lax.broadcasted_iota(jnp.int32, (S, TV), 1) + vbase
    onehot = jnp.where(idc_ref[0] == lane, 1.0, 0.0).astype(jnp.bfloat16)
    base = jax.lax.dot_general(
        proj_ref[0], embW_ref[...], (((1,), (1,)), ((), ())),
        preferred_element_type=jnp.float32)
    scat = jnp.dot(gated_ref[0], onehot, preferred_element_type=jnp.float32)
    out_ref[0] = base + scat + bias_ref[...]


def _vocab_call(proj, gated, emb_W, ids_col, bias2):
    B, S, E = proj.shape
    Vv = emb_W.shape[0]
    TV = 1280
    grid = (B, Vv // TV)
    return pl.pallas_call(
        _vocab_body,
        grid=grid,
        in_specs=[
            pl.BlockSpec((1, S, E), lambda b, v: (b, 0, 0)),
            pl.BlockSpec((1, S, S), lambda b, v: (b, 0, 0)),
            pl.BlockSpec((TV, E), lambda b, v: (v, 0)),
            pl.BlockSpec((1, S, 1), lambda b, v: (b, 0, 0)),
            pl.BlockSpec((1, TV), lambda b, v: (0, v)),
        ],
        out_specs=pl.BlockSpec((1, S, TV), lambda b, v: (b, 0, v)),
        out_shape=jax.ShapeDtypeStruct((B, S, Vv), jnp.float32),
        compiler_params=pltpu.CompilerParams(
            dimension_semantics=("parallel", "arbitrary"),
            vmem_limit_bytes=48 * 1024 * 1024,
        ),
        name="vocab_logits",
    )(proj, gated, emb_W, ids_col, bias2)


def kernel(input_ids, emb_W, W_ih, W_hh, b_ih, b_hh, in_res_W, q_W, q_b,
           k_W, k_b, gate_W, gate_b, h2e_W, h2e_b, out_bias,
           memory_scale, resid_lambda, input_lambda):
    B, S = input_ids.shape
    Hh = W_hh.shape[1]
    E = emb_W.shape[1]
    Mm = q_W.shape[0]

    ids_flat = input_ids.reshape(-1).astype(jnp.int32)
    x_proj2, inres2 = _embed_call(
        ids_flat, emb_W, W_ih.T, b_ih.reshape(1, -1), in_res_W.T)
    x_proj = x_proj2.reshape(B, S, 3 * Hh)
    inres = inres2.reshape(B, S, Hh)

    states = _gru_call(x_proj, W_hh.T.astype(jnp.bfloat16),
                       b_hh.reshape(1, -1))

    scal = jnp.stack([
        resid_lambda[0], input_lambda[0],
        jnp.asarray(memory_scale, jnp.float32), gate_b[0],
        jnp.asarray(1.0, jnp.float32) / jnp.sqrt(jnp.asarray(Mm, jnp.float32)),
    ]).reshape(1, 5)
    gated, proj = _attn_call(
        states, inres, q_W.T, q_b.reshape(1, -1), k_W.T, k_b.reshape(1, -1),
        gate_W, h2e_W.T, h2e_b.reshape(1, -1), scal)

    ids_col = input_ids[:, :, None].astype(jnp.int32)
    logits = _vocab_call(proj, gated, emb_W, ids_col, out_bias.reshape(1, -1))
    return logits


# speedup vs baseline: 9.2279x; 1.0544x over previous
import jax
import jax.numpy as jnp
from jax.experimental import pallas as pl
from jax.experimental.pallas import tpu as pltpu

_EPS_RMS = 1.1920929e-07
_F32_MIN = jnp.finfo(jnp.float32).min


def _embed_body(emb_hbm, ids_ref, wihT_ref, bih_ref, irT_ref,
                xp_ref, ir_ref, scr_ref, sem):
    tile = scr_ref.shape[0]
    base = pl.program_id(0) * tile

    def issue(i, carry):
        row = ids_ref[base + i]
        pltpu.make_async_copy(emb_hbm.at[row], scr_ref.at[i], sem).start()
        return carry

    jax.lax.fori_loop(0, tile, issue, 0)

    def drain(i, carry):
        pltpu.make_async_copy(emb_hbm.at[0], scr_ref.at[0], sem).wait()
        return carry

    jax.lax.fori_loop(0, tile, drain, 0)

    emb = scr_ref[...]
    xp_ref[...] = jnp.dot(emb, wihT_ref[...],
                          preferred_element_type=jnp.float32) + bih_ref[...]
    ir_ref[...] = jnp.dot(emb, irT_ref[...],
                          preferred_element_type=jnp.float32)


def _embed_call(ids_flat, emb_W, W_ihT, b_ih2, in_resT):
    n_tok = ids_flat.shape[0]
    E = emb_W.shape[1]
    H3 = W_ihT.shape[1]
    Hh = in_resT.shape[1]
    TA = 256
    grid = (n_tok // TA,)
    return pl.pallas_call(
        _embed_body,
        grid=grid,
        in_specs=[
            pl.BlockSpec(memory_space=pl.ANY),
            pl.BlockSpec(memory_space=pltpu.SMEM),
            pl.BlockSpec(memory_space=pltpu.VMEM),
            pl.BlockSpec(memory_space=pltpu.VMEM),
            pl.BlockSpec(memory_space=pltpu.VMEM),
        ],
        out_specs=[
            pl.BlockSpec((TA, H3), lambda g: (g, 0)),
            pl.BlockSpec((TA, Hh), lambda g: (g, 0)),
        ],
        out_shape=[
            jax.ShapeDtypeStruct((n_tok, H3), jnp.float32),
            jax.ShapeDtypeStruct((n_tok, Hh), jnp.float32),
        ],
        scratch_shapes=[
            pltpu.VMEM((TA, E), jnp.float32),
            pltpu.SemaphoreType.DMA,
        ],
        compiler_params=pltpu.CompilerParams(
            dimension_semantics=("parallel",),
            vmem_limit_bytes=40 * 1024 * 1024,
        ),
        name="embed_proj",
    )(emb_W, ids_flat, W_ihT, b_ih2, in_resT)


def _gru_body(xp_ref, whhT_ref, bhh_ref, st_ref):
    Hh = whhT_ref.shape[0]
    B, S = xp_ref.shape[0], xp_ref.shape[1]

    def step(t, h):
        hg = jnp.dot(h.astype(jnp.bfloat16), whhT_ref[...],
                     preferred_element_type=jnp.float32) + bhh_ref[...]
        xt = xp_ref[:, pl.ds(t, 1), :].reshape(B, 3 * Hh)
        r = jax.nn.sigmoid(xt[:, :Hh] + hg[:, :Hh])
        z = jax.nn.sigmoid(xt[:, Hh:2 * Hh] + hg[:, Hh:2 * Hh])
        n = jnp.tanh(xt[:, 2 * Hh:] + r * hg[:, 2 * Hh:])
        h_new = (1.0 - z) * n + z * h
        st_ref[:, pl.ds(t, 1), :] = h_new.reshape(B, 1, Hh)
        return h_new

    def step4(i, h):
        h = step(4 * i, h)
        h = step(4 * i + 1, h)
        h = step(4 * i + 2, h)
        return step(4 * i + 3, h)

    jax.lax.fori_loop(0, S // 4, step4, jnp.zeros((B, Hh), jnp.float32))


def _gru_call(x_proj, W_hhT, b_hh2):
    B, S, H3 = x_proj.shape
    Hh = W_hhT.shape[0]
    return pl.pallas_call(
        _gru_body,
        in_specs=[
            pl.BlockSpec(memory_space=pltpu.VMEM),
            pl.BlockSpec(memory_space=pltpu.VMEM),
            pl.BlockSpec(memory_space=pltpu.VMEM),
        ],
        out_specs=pl.BlockSpec(memory_space=pltpu.VMEM),
        out_shape=jax.ShapeDtypeStruct((B, S, Hh), jnp.float32),
        compiler_params=pltpu.CompilerParams(
            vmem_limit_bytes=52 * 1024 * 1024,
        ),
        name="gru_scan",
    )(x_proj, W_hhT, b_hh2)


def _attn_body(st_ref, ir_ref, qWT_ref, qb_ref, kWT_ref, kb_ref,
               gw_ref, h2eT_ref, h2eb_ref, scal_ref,
               gated_ref, proj_ref, mx_ref, q_scr, k_scr, s_scr):
    S = st_ref.shape[1]
    lam_r = scal_ref[0, 0]
    lam_i = scal_ref[0, 1]
    mscale = scal_ref[0, 2]
    gate_b = scal_ref[0, 3]
    inv_sqrt_m = scal_ref[0, 4]

    mx_ref[...] = lam_r * st_ref[0] + lam_i * ir_ref[0]
    ms = jnp.mean(mx_ref[...] * mx_ref[...], axis=-1, keepdims=True)
    mx_ref[...] = mx_ref[...] * jax.lax.rsqrt(ms + _EPS_RMS)

    q_scr[...] = jnp.dot(mx_ref[...], qWT_ref[...],
                         preferred_element_type=jnp.float32) + qb_ref[...]
    k_scr[...] = jnp.dot(mx_ref[...], kWT_ref[...],
                         preferred_element_type=jnp.float32) + kb_ref[...]
    proj_ref[0] = jnp.dot(mx_ref[...], h2eT_ref[...],
                          preferred_element_type=jnp.float32) + h2eb_ref[...]

    row = jax.lax.broadcasted_iota(jnp.int32, (S, S), 0)
    col = jax.lax.broadcasted_iota(jnp.int32, (S, S), 1)
    mask = col < row
    scores = jax.lax.dot_general(
        q_scr[...], k_scr[...], (((1,), (1,)), ((), ())),
        preferred_element_type=jnp.float32) * inv_sqrt_m
    s_scr[...] = jnp.where(mask, scores, _F32_MIN)

    m = jnp.max(s_scr[...], axis=-1, keepdims=True)
    row2 = jax.lax.broadcasted_iota(jnp.int32, (S, S), 0)
    col2 = jax.lax.broadcasted_iota(jnp.int32, (S, S), 1)
    s_scr[...] = jnp.exp(s_scr[...] - m) * jnp.where(col2 < row2, 1.0, 0.0)

    denom = jnp.sum(s_scr[...], axis=-1, keepdims=True)
    gpre = jnp.sum(mx_ref[...] * gw_ref[...], axis=-1, keepdims=True) + gate_b
    gate = jax.nn.sigmoid(gpre) * mscale
    gated_ref[0] = (s_scr[...] * (gate / jnp.maximum(denom, 1e-6))
                    ).astype(jnp.float8_e4m3fn)


def _attn_call(states, inres, qWT, qb2, kWT, kb2, gate_W, h2eT, h2eb2, scal):
    B, S, Hh = states.shape
    Mm = qWT.shape[1]
    E = h2eT.shape[1]
    return pl.pallas_call(
        _attn_body,
        grid=(B,),
        in_specs=[
            pl.BlockSpec((1, S, Hh), lambda b: (b, 0, 0)),
            pl.BlockSpec((1, S, Hh), lambda b: (b, 0, 0)),
            pl.BlockSpec(memory_space=pltpu.VMEM),
            pl.BlockSpec(memory_space=pltpu.VMEM),
            pl.BlockSpec(memory_space=pltpu.VMEM),
            pl.BlockSpec(memory_space=pltpu.VMEM),
            pl.BlockSpec(memory_space=pltpu.VMEM),
            pl.BlockSpec(memory_space=pltpu.VMEM),
            pl.BlockSpec(memory_space=pltpu.VMEM),
            pl.BlockSpec(memory_space=pltpu.SMEM),
        ],
        out_specs=[
            pl.BlockSpec((1, S, S), lambda b: (b, 0, 0)),
            pl.BlockSpec((1, S, E), lambda b: (b, 0, 0)),
        ],
        out_shape=[
            jax.ShapeDtypeStruct((B, S, S), jnp.float8_e4m3fn),
            jax.ShapeDtypeStruct((B, S, E), jnp.float32),
        ],
        scratch_shapes=[
            pltpu.VMEM((S, Hh), jnp.float32),
            pltpu.VMEM((S, Mm), jnp.float32),
            pltpu.VMEM((S, Mm), jnp.float32),
            pltpu.VMEM((S, S), jnp.float32),
        ],
        compiler_params=pltpu.CompilerParams(
            dimension_semantics=("parallel",),
            vmem_limit_bytes=52 * 1024 * 1024,
        ),
        name="mix_attn",
    )(states, inres, qWT, qb2, kWT, kb2, gate_W, h2eT, h2eb2, scal)


def _vocab_body(proj_ref, gated_ref, embW_ref, idc_ref, bias_ref, out_ref):
    S = gated_ref.shape[1]
    TV = embW_ref.shape[0]
    vbase = pl.program_id(1) * TV
    lane = jax.lax.broadcasted_iota(jnp.int32, (S, TV), 1) + vbase
    onehot = jnp.where(idc_ref[0] == lane, 1.0, 0.0).astype(jnp.float8_e4m3fn)
    base = jax.lax.dot_general(
        proj_ref[0], embW_ref[...], (((1,), (1,)), ((), ())),
        preferred_element_type=jnp.float32)
    scat = jnp.dot(gated_ref[0], onehot, preferred_element_type=jnp.float32)
    out_ref[0] = base + scat + bias_ref[...]


def _vocab_call(proj, gated, emb_W, ids_col, bias2):
    B, S, E = proj.shape
    Vv = emb_W.shape[0]
    TV = 1280
    grid = (B, Vv // TV)
    return pl.pallas_call(
        _vocab_body,
        grid=grid,
        in_specs=[
            pl.BlockSpec((1, S, E), lambda b, v: (b, 0, 0)),
            pl.BlockSpec((1, S, S), lambda b, v: (b, 0, 0)),
            pl.BlockSpec((TV, E), lambda b, v: (v, 0)),
            pl.BlockSpec((1, S, 1), lambda b, v: (b, 0, 0)),
            pl.BlockSpec((1, TV), lambda b, v: (0, v)),
        ],
        out_specs=pl.BlockSpec((1, S, TV), lambda b, v: (b, 0, v)),
        out_shape=jax.ShapeDtypeStruct((B, S, Vv), jnp.float32),
        compiler_params=pltpu.CompilerParams(
            dimension_semantics=("parallel", "arbitrary"),
            vmem_limit_bytes=48 * 1024 * 1024,
        ),
        name="vocab_logits",
    )(proj, gated, emb_W, ids_col, bias2)


def kernel(input_ids, emb_W, W_ih, W_hh, b_ih, b_hh, in_res_W, q_W, q_b,
           k_W, k_b, gate_W, gate_b, h2e_W, h2e_b, out_bias,
           memory_scale, resid_lambda, input_lambda):
    B, S = input_ids.shape
    Hh = W_hh.shape[1]
    E = emb_W.shape[1]
    Mm = q_W.shape[0]

    ids_flat = input_ids.reshape(-1).astype(jnp.int32)
    x_proj2, inres2 = _embed_call(
        ids_flat, emb_W, W_ih.T, b_ih.reshape(1, -1), in_res_W.T)
    x_proj = x_proj2.reshape(B, S, 3 * Hh)
    inres = inres2.reshape(B, S, Hh)

    states = _gru_call(x_proj, W_hh.T.astype(jnp.bfloat16),
                       b_hh.reshape(1, -1))

    scal = jnp.stack([
        resid_lambda[0], input_lambda[0],
        jnp.asarray(memory_scale, jnp.float32), gate_b[0],
        jnp.asarray(1.0, jnp.float32) / jnp.sqrt(jnp.asarray(Mm, jnp.float32)),
    ]).reshape(1, 5)
    gated, proj = _attn_call(
        states, inres, q_W.T, q_b.reshape(1, -1), k_W.T, k_b.reshape(1, -1),
        gate_W, h2e_W.T, h2e_b.reshape(1, -1), scal)

    ids_col = input_ids[:, :, None].astype(jnp.int32)
    logits = _vocab_call(proj, gated, emb_W, ids_col, out_bias.reshape(1, -1))
    return logits


# speedup vs baseline: 9.3412x; 1.0123x over previous
import jax
import jax.numpy as jnp
from jax.experimental import pallas as pl
from jax.experimental.pallas import tpu as pltpu

_EPS_RMS = 1.1920929e-07
_F32_MIN = jnp.finfo(jnp.float32).min


def _embed_body(emb_hbm, ids_ref, wihT_ref, bih_ref, irT_ref,
                xp_ref, ir_ref, scr_ref, sem):
    tile = scr_ref.shape[0]
    base = pl.program_id(0) * tile

    def issue(i, carry):
        row = ids_ref[base + i]
        pltpu.make_async_copy(emb_hbm.at[row], scr_ref.at[i], sem).start()
        return carry

    jax.lax.fori_loop(0, tile, issue, 0)

    def drain(i, carry):
        pltpu.make_async_copy(emb_hbm.at[0], scr_ref.at[0], sem).wait()
        return carry

    jax.lax.fori_loop(0, tile, drain, 0)

    emb = scr_ref[...]
    xp_ref[...] = jnp.dot(emb, wihT_ref[...],
                          preferred_element_type=jnp.float32) + bih_ref[...]
    ir_ref[...] = jnp.dot(emb, irT_ref[...],
                          preferred_element_type=jnp.float32)


def _embed_call(ids_flat, emb_W, W_ihT, b_ih2, in_resT):
    n_tok = ids_flat.shape[0]
    E = emb_W.shape[1]
    H3 = W_ihT.shape[1]
    Hh = in_resT.shape[1]
    TA = 256
    grid = (n_tok // TA,)
    return pl.pallas_call(
        _embed_body,
        grid=grid,
        in_specs=[
            pl.BlockSpec(memory_space=pl.ANY),
            pl.BlockSpec(memory_space=pltpu.SMEM),
            pl.BlockSpec(memory_space=pltpu.VMEM),
            pl.BlockSpec(memory_space=pltpu.VMEM),
            pl.BlockSpec(memory_space=pltpu.VMEM),
        ],
        out_specs=[
            pl.BlockSpec((TA, H3), lambda g: (g, 0)),
            pl.BlockSpec((TA, Hh), lambda g: (g, 0)),
        ],
        out_shape=[
            jax.ShapeDtypeStruct((n_tok, H3), jnp.float32),
            jax.ShapeDtypeStruct((n_tok, Hh), jnp.float32),
        ],
        scratch_shapes=[
            pltpu.VMEM((TA, E), jnp.float32),
            pltpu.SemaphoreType.DMA,
        ],
        compiler_params=pltpu.CompilerParams(
            dimension_semantics=("parallel",),
            vmem_limit_bytes=40 * 1024 * 1024,
        ),
        name="embed_proj",
    )(emb_W, ids_flat, W_ihT, b_ih2, in_resT)


def _gru_body(xp_ref, whhT_ref, bhh_ref, st_ref):
    Hh = whhT_ref.shape[0]
    B, S = xp_ref.shape[0], xp_ref.shape[1]

    def step(t, h):
        hg = jnp.dot(h.astype(jnp.bfloat16), whhT_ref[...],
                     preferred_element_type=jnp.float32) + bhh_ref[...]
        xt = xp_ref[:, pl.ds(t, 1), :].reshape(B, 3 * Hh)
        r = jax.nn.sigmoid(xt[:, :Hh] + hg[:, :Hh])
        z = jax.nn.sigmoid(xt[:, Hh:2 * Hh] + hg[:, Hh:2 * Hh])
        n = jnp.tanh(xt[:, 2 * Hh:] + r * hg[:, 2 * Hh:])
        h_new = (1.0 - z) * n + z * h
        st_ref[:, pl.ds(t, 1), :] = h_new.reshape(B, 1, Hh)
        return h_new

    def step8(i, h):
        for u in range(8):
            h = step(8 * i + u, h)
        return h

    jax.lax.fori_loop(0, S // 8, step8, jnp.zeros((B, Hh), jnp.float32))


def _gru_call(x_proj, W_hhT, b_hh2):
    B, S, H3 = x_proj.shape
    Hh = W_hhT.shape[0]
    return pl.pallas_call(
        _gru_body,
        in_specs=[
            pl.BlockSpec(memory_space=pltpu.VMEM),
            pl.BlockSpec(memory_space=pltpu.VMEM),
            pl.BlockSpec(memory_space=pltpu.VMEM),
        ],
        out_specs=pl.BlockSpec(memory_space=pltpu.VMEM),
        out_shape=jax.ShapeDtypeStruct((B, S, Hh), jnp.float32),
        compiler_params=pltpu.CompilerParams(
            vmem_limit_bytes=52 * 1024 * 1024,
        ),
        name="gru_scan",
    )(x_proj, W_hhT, b_hh2)


def _attn_body(st_ref, ir_ref, qWT_ref, qb_ref, kWT_ref, kb_ref,
               gw_ref, h2eT_ref, h2eb_ref, scal_ref,
               gated_ref, proj_ref, mx_ref, q_scr, k_scr, s_scr):
    S = st_ref.shape[1]
    lam_r = scal_ref[0, 0]
    lam_i = scal_ref[0, 1]
    mscale = scal_ref[0, 2]
    gate_b = scal_ref[0, 3]
    inv_sqrt_m = scal_ref[0, 4]

    mx_ref[...] = lam_r * st_ref[0] + lam_i * ir_ref[0]
    ms = jnp.mean(mx_ref[...] * mx_ref[...], axis=-1, keepdims=True)
    mx_ref[...] = mx_ref[...] * jax.lax.rsqrt(ms + _EPS_RMS)

    q_scr[...] = jnp.dot(mx_ref[...], qWT_ref[...],
                         preferred_element_type=jnp.float32) + qb_ref[...]
    k_scr[...] = jnp.dot(mx_ref[...], kWT_ref[...],
                         preferred_element_type=jnp.float32) + kb_ref[...]
    proj_ref[0] = jnp.dot(mx_ref[...], h2eT_ref[...],
                          preferred_element_type=jnp.float32) + h2eb_ref[...]

    row = jax.lax.broadcasted_iota(jnp.int32, (S, S), 0)
    col = jax.lax.broadcasted_iota(jnp.int32, (S, S), 1)
    mask = col < row
    scores = jax.lax.dot_general(
        q_scr[...], k_scr[...], (((1,), (1,)), ((), ())),
        preferred_element_type=jnp.float32) * inv_sqrt_m
    s_scr[...] = jnp.where(mask, scores, _F32_MIN)

    m = jnp.max(s_scr[...], axis=-1, keepdims=True)
    row2 = jax.lax.broadcasted_iota(jnp.int32, (S, S), 0)
    col2 = jax.lax.broadcasted_iota(jnp.int32, (S, S), 1)
    s_scr[...] = jnp.exp(s_scr[...] - m) * jnp.where(col2 < row2, 1.0, 0.0)

    denom = jnp.sum(s_scr[...], axis=-1, keepdims=True)
    gpre = jnp.sum(mx_ref[...] * gw_ref[...], axis=-1, keepdims=True) + gate_b
    gate = jax.nn.sigmoid(gpre) * mscale
    gated_ref[0] = (s_scr[...] * (gate / jnp.maximum(denom, 1e-6))
                    ).astype(jnp.float8_e4m3fn)


def _attn_call(states, inres, qWT, qb2, kWT, kb2, gate_W, h2eT, h2eb2, scal):
    B, S, Hh = states.shape
    Mm = qWT.shape[1]
    E = h2eT.shape[1]
    return pl.pallas_call(
        _attn_body,
        grid=(B,),
        in_specs=[
            pl.BlockSpec((1, S, Hh), lambda b: (b, 0, 0)),
            pl.BlockSpec((1, S, Hh), lambda b: (b, 0, 0)),
            pl.BlockSpec(memory_space=pltpu.VMEM),
            pl.BlockSpec(memory_space=pltpu.VMEM),
            pl.BlockSpec(memory_space=pltpu.VMEM),
            pl.BlockSpec(memory_space=pltpu.VMEM),
            pl.BlockSpec(memory_space=pltpu.VMEM),
            pl.BlockSpec(memory_space=pltpu.VMEM),
            pl.BlockSpec(memory_space=pltpu.VMEM),
            pl.BlockSpec(memory_space=pltpu.SMEM),
        ],
        out_specs=[
            pl.BlockSpec((1, S, S), lambda b: (b, 0, 0)),
            pl.BlockSpec((1, S, E), lambda b: (b, 0, 0)),
        ],
        out_shape=[
            jax.ShapeDtypeStruct((B, S, S), jnp.float8_e4m3fn),
            jax.ShapeDtypeStruct((B, S, E), jnp.float32),
        ],
        scratch_shapes=[
            pltpu.VMEM((S, Hh), jnp.float32),
            pltpu.VMEM((S, Mm), jnp.float32),
            pltpu.VMEM((S, Mm), jnp.float32),
            pltpu.VMEM((S, S), jnp.float32),
        ],
        compiler_params=pltpu.CompilerParams(
            dimension_semantics=("parallel",),
            vmem_limit_bytes=52 * 1024 * 1024,
        ),
        name="mix_attn",
    )(states, inres, qWT, qb2, kWT, kb2, gate_W, h2eT, h2eb2, scal)


def _vocab_body(proj_ref, gated_ref, embW_ref, idc_ref, bias_ref, out_ref):
    S = gated_ref.shape[1]
    TV = embW_ref.shape[0]
    vbase = pl.program_id(1) * TV
    lane = jax.lax.broadcasted_iota(jnp.int32, (S, TV), 1) + vbase
    onehot = jnp.where(idc_ref[0] == lane, 1.0, 0.0).astype(jnp.float8_e4m3fn)
    base = jax.lax.dot_general(
        proj_ref[0], embW_ref[...], (((1,), (1,)), ((), ())),
        preferred_element_type=jnp.float32)
    scat = jnp.dot(gated_ref[0], onehot, preferred_element_type=jnp.float32)
    out_ref[0] = base + scat + bias_ref[...]


def _vocab_call(proj, gated, emb_W, ids_col, bias2):
    B, S, E = proj.shape
    Vv = emb_W.shape[0]
    TV = 3200
    grid = (B, Vv // TV)
    return pl.pallas_call(
        _vocab_body,
        grid=grid,
        in_specs=[
            pl.BlockSpec((1, S, E), lambda b, v: (b, 0, 0)),
            pl.BlockSpec((1, S, S), lambda b, v: (b, 0, 0)),
            pl.BlockSpec((TV, E), lambda b, v: (v, 0)),
            pl.BlockSpec((1, S, 1), lambda b, v: (b, 0, 0)),
            pl.BlockSpec((1, TV), lambda b, v: (0, v)),
        ],
        out_specs=pl.BlockSpec((1, S, TV), lambda b, v: (b, 0, v)),
        out_shape=jax.ShapeDtypeStruct((B, S, Vv), jnp.float32),
        compiler_params=pltpu.CompilerParams(
            dimension_semantics=("parallel", "arbitrary"),
            vmem_limit_bytes=58 * 1024 * 1024,
        ),
        name="vocab_logits",
    )(proj, gated, emb_W, ids_col, bias2)


def kernel(input_ids, emb_W, W_ih, W_hh, b_ih, b_hh, in_res_W, q_W, q_b,
           k_W, k_b, gate_W, gate_b, h2e_W, h2e_b, out_bias,
           memory_scale, resid_lambda, input_lambda):
    B, S = input_ids.shape
    Hh = W_hh.shape[1]
    E = emb_W.shape[1]
    Mm = q_W.shape[0]

    ids_flat = input_ids.reshape(-1).astype(jnp.int32)
    x_proj2, inres2 = _embed_call(
        ids_flat, emb_W, W_ih.T, b_ih.reshape(1, -1), in_res_W.T)
    x_proj = x_proj2.reshape(B, S, 3 * Hh)
    inres = inres2.reshape(B, S, Hh)

    states = _gru_call(x_proj, W_hh.T.astype(jnp.bfloat16),
                       b_hh.reshape(1, -1))

    scal = jnp.stack([
        resid_lambda[0], input_lambda[0],
        jnp.asarray(memory_scale, jnp.float32), gate_b[0],
        jnp.asarray(1.0, jnp.float32) / jnp.sqrt(jnp.asarray(Mm, jnp.float32)),
    ]).reshape(1, 5)
    gated, proj = _attn_call(
        states, inres, q_W.T, q_b.reshape(1, -1), k_W.T, k_b.reshape(1, -1),
        gate_W, h2e_W.T, h2e_b.reshape(1, -1), scal)

    ids_col = input_ids[:, :, None].astype(jnp.int32)
    logits = _vocab_call(proj, gated, emb_W, ids_col, out_bias.reshape(1, -1))
    return logits
